# baseline (speedup 1.0000x reference)
import functools

import jax
import jax.numpy as jnp
from jax import lax
from jax.experimental import pallas as pl
from jax.experimental.pallas import tpu as pltpu
from jax.experimental.pallas import tpu_sc as plsc

_K = 64
_R = 64
_C = 8192
_NVEC = _C // 16


def _stage_a_body(s_ref, u_ref, sn_ref, key_ref):
    S = s_ref[...]
    u = u_ref[...]
    m = jnp.max(S, axis=1, keepdims=True)
    e = jnp.exp(S - m)
    denom = jnp.sum(e, axis=1, keepdims=True)
    Sn = e / denom
    g = -jnp.log(-jnp.log(u + 1e-10) + 1e-10)
    P = Sn + g
    P = jnp.where(P == 0.0, 0.0, P)
    b = lax.bitcast_convert_type(P, jnp.uint32)
    neg = b >= jnp.uint32(0x80000000)
    key = jnp.where(neg, ~b, b | jnp.uint32(0x80000000))
    sn_ref[...] = Sn
    key_ref[...] = key


def _stage_a(S, u):
    return pl.pallas_call(
        _stage_a_body,
        out_shape=(
            jax.ShapeDtypeStruct((_R, _C), jnp.float32),
            jax.ShapeDtypeStruct((_R, _C), jnp.uint32),
        ),
    )(S, u)


def _digit_cut(totals_ref, kk):
    def gloop(g2, carry):
        cnt_cond, carry_after = carry
        g = 15 - g2
        t = totals_ref[pl.ds(g * 16, 16)]
        ssum = lax.rev(plsc.cumsum(lax.rev(t, (0,))), (0,)) + carry_after
        cnt_cond = cnt_cond + jnp.sum((ssum >= kk).astype(jnp.int32))
        carry_after = carry_after + jnp.sum(t)
        return cnt_cond, carry_after

    cnt_cond, _ = lax.fori_loop(0, 16, gloop, (jnp.int32(0), jnp.int32(0)))
    D = cnt_cond - 1

    def cloop(g, acc):
        t = totals_ref[pl.ds(g * 16, 16)]
        dig = lax.iota(jnp.int32, 16) + g * 16
        return acc + jnp.sum(jnp.where(dig > D, t, 0))

    c_gt = lax.fori_loop(0, 16, cloop, jnp.int32(0))
    return D, c_gt


def _reduce_hist(hist_ref, totals_ref):
    zero16 = jnp.zeros((16,), jnp.int32)

    @plsc.parallel_loop(0, 16)
    def gloop(g):
        acc = zero16
        for l in range(16):
            acc = acc + hist_ref[pl.ds(l * 256 + g * 16, 16)]
            hist_ref[pl.ds(l * 256 + g * 16, 16)] = zero16
        totals_ref[pl.ds(g * 16, 16)] = acc


def _sc_select_body(key_hbm, sn_hbm, out_hbm, key_v, sn_v, cand_k, cand_i,
                    hist, totals, sn_sem, out_sem):
    nc = 2
    wid = lax.axis_index("s") * nc + lax.axis_index("c")
    lane = lax.iota(jnp.int32, 16)
    lane_base = lane * 256
    ones = jnp.ones((16,), jnp.int32)

    @plsc.parallel_loop(0, 256, unroll=8)
    def zloop(i):
        hist[pl.ds(i * 16, 16)] = jnp.zeros((16,), jnp.int32)

    out_cp = None
    for r in range(2):
        row = wid * 2 + r
        pltpu.sync_copy(key_hbm.at[row], key_v)
        if out_cp is not None:
            out_cp.wait()
        sn_cp = pltpu.async_copy(sn_hbm.at[row], sn_v, sn_sem)

        @plsc.parallel_loop(0, _NVEC, unroll=16)
        def h1(i):
            k = key_v[pl.ds(i * 16, 16)]
            d = lax.shift_right_logical(k, jnp.uint32(24)).astype(jnp.int32)
            plsc.addupdate_scatter(hist, [lane_base + d], ones)
        _reduce_hist(hist, totals)
        D1, c_gt1 = _digit_cut(totals, jnp.int32(_K))

        @plsc.parallel_loop(0, _NVEC, unroll=16)
        def h2(i):
            k = key_v[pl.ds(i * 16, 16)]
            d1 = lax.shift_right_logical(k, jnp.uint32(24)).astype(jnp.int32)
            m = d1 == D1
            d2 = (lax.shift_right_logical(k, jnp.uint32(16)).astype(jnp.int32)
                  & 255)
            plsc.addupdate_scatter(hist, [lane_base + d2], ones, mask=m)
        _reduce_hist(hist, totals)
        D2, dgt2 = _digit_cut(totals, _K - c_gt1)
        c_gt2 = c_gt1 + dgt2
        P2 = D1 * 256 + D2

        @plsc.parallel_loop(0, _NVEC, unroll=16, carry=jnp.int32(0))
        def comp(i, off):
            k = key_v[pl.ds(i * 16, 16)]
            top16 = lax.shift_right_logical(k, jnp.uint32(16)).astype(jnp.int32)
            m = top16 == P2
            mi = m.astype(jnp.int32)
            rank = plsc.cumsum(mi)
            pos = off + rank - 1
            plsc.store_scatter(cand_k, [pos], plsc.bitcast(k, jnp.int32),
                               mask=m)
            plsc.store_scatter(cand_i, [pos], lane + i * 16, mask=m)
            return off + jnp.sum(mi)

        ncand = comp
        trip = (ncand + 15) // 16
        kk3 = _K - c_gt2

        def bitstep(b, t16):
            cand = t16 | (jnp.int32(1) << (jnp.int32(15) - b))

            def cnt_loop(j, c):
                ck = cand_k[pl.ds(j * 16, 16)]
                low = ck & jnp.int32(0xFFFF)
                valid = (lane + j * 16) < ncand
                return c + jnp.sum((valid & (low >= cand)).astype(jnp.int32))

            c = lax.fori_loop(0, trip, cnt_loop, jnp.int32(0))
            return jnp.where(c >= kk3, cand, t16)

        T16 = lax.fori_loop(0, 16, bitstep, jnp.int32(0))

        def gtloop(j, c):
            ck = cand_k[pl.ds(j * 16, 16)]
            low = ck & jnp.int32(0xFFFF)
            valid = (lane + j * 16) < ncand
            return c + jnp.sum((valid & (low > T16)).astype(jnp.int32))

        c_gt = c_gt2 + lax.fori_loop(0, trip, gtloop, jnp.int32(0))
        need = _K - c_gt

        def tie(j, carry):
            cnt_eq, jacc = carry
            ck = cand_k[pl.ds(j * 16, 16)]
            low = ck & jnp.int32(0xFFFF)
            valid = (lane + j * 16) < ncand
            m = valid & (low == T16)
            mi = m.astype(jnp.int32)
            grank = cnt_eq + plsc.cumsum(mi)
            hit = m & (grank == need)
            ci = cand_i[pl.ds(j * 16, 16)]
            jacc = jacc + jnp.sum(jnp.where(hit, ci, 0))
            return cnt_eq + jnp.sum(mi), jacc

        _, J = lax.fori_loop(0, trip, tie, (jnp.int32(0), jnp.int32(0)))

        T = (lax.shift_left(P2.astype(jnp.uint32), jnp.uint32(16))
             | T16.astype(jnp.uint32))
        sn_cp.wait()

        @plsc.parallel_loop(0, _NVEC, unroll=16)
        def apply(i):
            k = key_v[pl.ds(i * 16, 16)]
            sv = sn_v[pl.ds(i * 16, 16)]
            idx = lane + i * 16
            m = (k > T) | ((k == T) & (idx <= J))
            sn_v[pl.ds(i * 16, 16)] = jnp.where(m, sv, 0.0)

        out_cp = pltpu.async_copy(sn_v, out_hbm.at[row], out_sem)

    out_cp.wait()


def _sc_select(key, Sn):
    mesh = plsc.VectorSubcoreMesh(core_axis_name="c", subcore_axis_name="s")
    f = functools.partial(
        pl.kernel,
        out_type=jax.ShapeDtypeStruct((_R, _C), jnp.float32),
        mesh=mesh,
        compiler_params=pltpu.CompilerParams(needs_layout_passes=False),
        scratch_types=[
            pltpu.VMEM((_C,), jnp.uint32),
            pltpu.VMEM((_C,), jnp.float32),
            pltpu.VMEM((_C,), jnp.int32),
            pltpu.VMEM((_C,), jnp.int32),
            pltpu.VMEM((4096,), jnp.int32),
            pltpu.VMEM((256,), jnp.int32),
            pltpu.SemaphoreType.DMA,
            pltpu.SemaphoreType.DMA,
        ],
    )(_sc_select_body)
    return f(key, Sn)


def kernel(S, u):
    Sn, key = _stage_a(S, u)
    return _sc_select(key, Sn)


# speedup vs baseline: 1.2082x; 1.2082x over previous
import functools

import jax
import jax.numpy as jnp
from jax import lax
from jax.experimental import pallas as pl
from jax.experimental.pallas import tpu as pltpu
from jax.experimental.pallas import tpu_sc as plsc

_K = 64
_R = 64
_C = 8192
_NVEC = _C // 16


def _stage_a_body(s_ref, u_ref, sn_ref, key_ref):
    S = s_ref[...]
    u = u_ref[...]
    m = jnp.max(S, axis=1, keepdims=True)
    e = jnp.exp(S - m)
    denom = jnp.sum(e, axis=1, keepdims=True)
    Sn = e / denom
    g = -jnp.log(-jnp.log(u + 1e-10) + 1e-10)
    P = Sn + g
    P = jnp.where(P == 0.0, 0.0, P)
    b = lax.bitcast_convert_type(P, jnp.uint32)
    neg = b >= jnp.uint32(0x80000000)
    key = jnp.where(neg, ~b, b | jnp.uint32(0x80000000))
    sn_ref[...] = Sn
    key_ref[...] = key


def _stage_a(S, u):
    return pl.pallas_call(
        _stage_a_body,
        out_shape=(
            jax.ShapeDtypeStruct((_R, _C), jnp.float32),
            jax.ShapeDtypeStruct((_R, _C), jnp.uint32),
        ),
    )(S, u)


def _digit_cut(totals_ref, kk):
    def gloop(g2, carry):
        cnt_cond, carry_after = carry
        g = 15 - g2
        t = totals_ref[pl.ds(g * 16, 16)]
        ssum = lax.rev(plsc.cumsum(lax.rev(t, (0,))), (0,)) + carry_after
        cnt_cond = cnt_cond + jnp.sum((ssum >= kk).astype(jnp.int32))
        carry_after = carry_after + jnp.sum(t)
        return cnt_cond, carry_after

    cnt_cond, _ = lax.fori_loop(0, 16, gloop, (jnp.int32(0), jnp.int32(0)))
    D = cnt_cond - 1

    def cloop(g, acc):
        t = totals_ref[pl.ds(g * 16, 16)]
        dig = lax.iota(jnp.int32, 16) + g * 16
        return acc + jnp.sum(jnp.where(dig > D, t, 0))

    c_gt = lax.fori_loop(0, 16, cloop, jnp.int32(0))
    return D, c_gt


def _reduce_hist(hist_ref, totals_ref):
    zero16 = jnp.zeros((16,), jnp.int32)

    @plsc.parallel_loop(0, 16)
    def gloop(g):
        acc = zero16
        for l in range(16):
            acc = acc + hist_ref[pl.ds(l * 256 + g * 16, 16)]
            hist_ref[pl.ds(l * 256 + g * 16, 16)] = zero16
        totals_ref[pl.ds(g * 16, 16)] = acc


def _sc_select_body(key_hbm, sn_hbm, out_hbm, key_v, sn_v, cand_k, cand_i,
                    hist, totals, sn_sem, out_sem):
    nc = 2
    wid = lax.axis_index("s") * nc + lax.axis_index("c")
    lane = lax.iota(jnp.int32, 16)
    lane_base = lane * 256
    ones = jnp.ones((16,), jnp.int32)

    @plsc.parallel_loop(0, 256, unroll=8)
    def zloop(i):
        hist[pl.ds(i * 16, 16)] = jnp.zeros((16,), jnp.int32)

    out_cp = None
    for r in range(2):
        row = wid * 2 + r
        pltpu.sync_copy(key_hbm.at[row], key_v)
        if out_cp is not None:
            out_cp.wait()
        sn_cp = pltpu.async_copy(sn_hbm.at[row], sn_v, sn_sem)

        @plsc.parallel_loop(0, _NVEC, unroll=16)
        def h1(i):
            k = key_v[pl.ds(i * 16, 16)]
            d = lax.shift_right_logical(k, jnp.uint32(24)).astype(jnp.int32)
            plsc.addupdate_scatter(hist, [lane_base + d], ones)
        _reduce_hist(hist, totals)
        D1, c_gt1 = _digit_cut(totals, jnp.int32(_K))

        @plsc.parallel_loop(0, _NVEC, unroll=16)
        def h2(i):
            k = key_v[pl.ds(i * 16, 16)]
            d1 = lax.shift_right_logical(k, jnp.uint32(24)).astype(jnp.int32)
            m = d1 == D1
            d2 = (lax.shift_right_logical(k, jnp.uint32(16)).astype(jnp.int32)
                  & 255)
            plsc.addupdate_scatter(hist, [lane_base + d2], ones, mask=m)
        _reduce_hist(hist, totals)
        D2, dgt2 = _digit_cut(totals, _K - c_gt1)
        c_gt2 = c_gt1 + dgt2
        P2 = D1 * 256 + D2

        @plsc.parallel_loop(0, _NVEC, unroll=8, carry=jnp.int32(0))
        def comp(i, off):
            k = key_v[pl.ds(i * 16, 16)]
            top16 = lax.shift_right_logical(k, jnp.uint32(16)).astype(jnp.int32)
            m = top16 == P2
            mi = m.astype(jnp.int32)
            rank = plsc.cumsum(mi)
            pos = off + rank - 1
            plsc.store_scatter(cand_k, [pos], plsc.bitcast(k, jnp.int32),
                               mask=m)
            plsc.store_scatter(cand_i, [pos], lane + i * 16, mask=m)
            return off + jnp.sum(mi)

        ncand = comp
        trip = (ncand + 15) // 16
        kk3 = _K - c_gt2

        def bitstep(b, t16):
            cand = t16 | (jnp.int32(1) << (jnp.int32(15) - b))

            def cnt_loop(j, c):
                ck = cand_k[pl.ds(j * 16, 16)]
                low = ck & jnp.int32(0xFFFF)
                valid = (lane + j * 16) < ncand
                return c + jnp.sum((valid & (low >= cand)).astype(jnp.int32))

            c = lax.fori_loop(0, trip, cnt_loop, jnp.int32(0))
            return jnp.where(c >= kk3, cand, t16)

        T16 = lax.fori_loop(0, 16, bitstep, jnp.int32(0))

        def gtloop(j, c):
            ck = cand_k[pl.ds(j * 16, 16)]
            low = ck & jnp.int32(0xFFFF)
            valid = (lane + j * 16) < ncand
            return c + jnp.sum((valid & (low > T16)).astype(jnp.int32))

        c_gt = c_gt2 + lax.fori_loop(0, trip, gtloop, jnp.int32(0))
        need = _K - c_gt

        def tie(j, carry):
            cnt_eq, jacc = carry
            ck = cand_k[pl.ds(j * 16, 16)]
            low = ck & jnp.int32(0xFFFF)
            valid = (lane + j * 16) < ncand
            m = valid & (low == T16)
            mi = m.astype(jnp.int32)
            grank = cnt_eq + plsc.cumsum(mi)
            hit = m & (grank == need)
            ci = cand_i[pl.ds(j * 16, 16)]
            jacc = jacc + jnp.sum(jnp.where(hit, ci, 0))
            return cnt_eq + jnp.sum(mi), jacc

        _, J = lax.fori_loop(0, trip, tie, (jnp.int32(0), jnp.int32(0)))

        T = (lax.shift_left(P2.astype(jnp.uint32), jnp.uint32(16))
             | T16.astype(jnp.uint32))
        sn_cp.wait()

        @plsc.parallel_loop(0, _NVEC, unroll=8)
        def apply(i):
            k = key_v[pl.ds(i * 16, 16)]
            sv = sn_v[pl.ds(i * 16, 16)]
            idx = lane + i * 16
            m = (k > T) | ((k == T) & (idx <= J))
            sn_v[pl.ds(i * 16, 16)] = jnp.where(m, sv, 0.0)

        out_cp = pltpu.async_copy(sn_v, out_hbm.at[row], out_sem)

    out_cp.wait()


def _sc_select(key, Sn):
    mesh = plsc.VectorSubcoreMesh(core_axis_name="c", subcore_axis_name="s")
    f = functools.partial(
        pl.kernel,
        out_type=jax.ShapeDtypeStruct((_R, _C), jnp.float32),
        mesh=mesh,
        compiler_params=pltpu.CompilerParams(needs_layout_passes=False),
        scratch_types=[
            pltpu.VMEM((_C,), jnp.uint32),
            pltpu.VMEM((_C,), jnp.float32),
            pltpu.VMEM((_C,), jnp.int32),
            pltpu.VMEM((_C,), jnp.int32),
            pltpu.VMEM((4096,), jnp.int32),
            pltpu.VMEM((256,), jnp.int32),
            pltpu.SemaphoreType.DMA,
            pltpu.SemaphoreType.DMA,
        ],
    )(_sc_select_body)
    return f(key, Sn)


def kernel(S, u):
    Sn, key = _stage_a(S, u)
    return _sc_select(key, Sn)
